# baseline (speedup 1.0000x reference)
import functools

import jax
import jax.numpy as jnp
from jax import lax
from jax.experimental import pallas as pl
from jax.experimental.pallas import tpu as pltpu
from jax.experimental.pallas import tpu_sc as plsc

MAX_POS = 1024
DIM = 64

try:
    _INFO = plsc.get_sparse_core_info()
    NC, NS, L = _INFO.num_cores, _INFO.num_subcores, _INFO.num_lanes
except Exception:
    NC, NS, L = 2, 16, 16
NW = NC * NS

GBLK = 128
SPLITS = ((0, 1024, 32), (1024, 4096, 96))


def _table_body(x_ref, y_ref, w_ref, b_ref, t_ref):
    wt = w_ref[...].T
    bias = b_ref[...] * 0.25
    t_ref[0:MAX_POS, :] = (
        jnp.dot(x_ref[...], wt, preferred_element_type=jnp.float32) + bias
    ).astype(jnp.bfloat16)
    t_ref[MAX_POS : 2 * MAX_POS, :] = (
        jnp.dot(y_ref[...], wt, preferred_element_type=jnp.float32) + bias
    ).astype(jnp.bfloat16)


def _build_table(x_table, y_table, W, b):
    return pl.pallas_call(
        _table_body,
        out_shape=jax.ShapeDtypeStruct((2 * MAX_POS, DIM), jnp.bfloat16),
    )(x_table, y_table, W, b.reshape(1, DIM))


def _sc_body(
    rows_total,
    CHUNK,
    t_hbm,
    bb_hbm,
    out_hbm,
    bb_v,
    idx_v,
    rows_v,
    out_v,
    sem0,
    sem1,
    semw0,
    semw1,
):
    rw = rows_total // NW
    n_chunks = rw // CHUNK
    wid = lax.axis_index("s") * NC + lax.axis_index("c")
    base_row = wid * rw
    sems = (sem0, sem1)
    semws = (semw0, semw1)
    nj = 4 * CHUNK // GBLK

    offs = (lax.iota(jnp.int32, L) % 2) * MAX_POS

    def stage_and_fire(c, p):
        row0 = base_row + c * CHUNK
        pltpu.sync_copy(bb_hbm.at[pl.ds(row0 * 4, 4 * CHUNK)], bb_v)
        for j in range(nj):
            for i in range(GBLK // L):
                v = bb_v[pl.ds(j * GBLK + i * L, L)]
                f = jnp.clip(v * float(MAX_POS), 0.0, float(MAX_POS - 1))
                idx_v[p, j, pl.ds(i * L, L)] = f.astype(jnp.int32) + offs
        for j in range(nj):
            pltpu.async_copy(
                t_hbm.at[idx_v.at[p, j]],
                rows_v.at[p, pl.ds(j * GBLK, GBLK)],
                sems[p],
            )

    def consume(c, p, k):
        row0 = base_row + c * CHUNK
        for j in range(nj):
            pltpu.make_async_copy(
                t_hbm.at[idx_v.at[p, j]],
                rows_v.at[p, pl.ds(j * GBLK, GBLK)],
                sems[p],
            ).wait()

        @pl.when(k > 0)
        def _():
            prev0 = base_row + (c - 2) * CHUNK
            pltpu.make_async_copy(
                out_v.at[p], out_hbm.at[pl.ds(prev0, CHUNK)], semws[p]
            ).wait()

        @plsc.parallel_loop(0, CHUNK, 1, unroll=8)
        def sum_body(r):
            for h in range(DIM // (2 * L)):
                ds = pl.ds(h * 2 * L, 2 * L)
                a = [None] * 4
                bvs = [None] * 4
                for i in range(4):
                    a[i], bvs[i] = plsc.unpack(
                        rows_v[p, 4 * r + i, ds], format=plsc.PackFormat.INTERLEAVED
                    )
                sa = (a[0] + a[1]) + (a[2] + a[3])
                sb = (bvs[0] + bvs[1]) + (bvs[2] + bvs[3])
                out_v[p, r, pl.ds(h * 2 * L, L)] = jnp.maximum(sa, 0.0)
                out_v[p, r, pl.ds(h * 2 * L + L, L)] = jnp.maximum(sb, 0.0)

        pltpu.async_copy(out_v.at[p], out_hbm.at[pl.ds(row0, CHUNK)], semws[p])

    stage_and_fire(0, 0)

    def pair_body(k, carry):
        for p in range(2):
            c = 2 * k + p

            @pl.when(c + 1 < n_chunks)
            def _():
                stage_and_fire(c + 1, 1 - p)

            consume(c, p, k)
        return carry

    lax.fori_loop(0, n_chunks // 2, pair_body, 0)

    for p in range(2):
        last = base_row + (n_chunks - 2 + p) * CHUNK
        pltpu.make_async_copy(
            out_v.at[p], out_hbm.at[pl.ds(last, CHUNK)], semws[p]
        ).wait()


def _lookup(t, bb_flat, rows_total, CHUNK):
    mesh = plsc.VectorSubcoreMesh(
        core_axis_name="c", subcore_axis_name="s", num_cores=NC, num_subcores=NS
    )
    f = pl.kernel(
        functools.partial(_sc_body, rows_total, CHUNK),
        out_type=jax.ShapeDtypeStruct((rows_total, DIM), jnp.float32),
        mesh=mesh,
        scratch_types=[
            pltpu.VMEM((4 * CHUNK,), jnp.float32),
            pltpu.VMEM((2, 4 * CHUNK // GBLK, GBLK), jnp.int32),
            pltpu.VMEM((2, 4 * CHUNK, DIM), jnp.bfloat16),
            pltpu.VMEM((2, CHUNK, DIM), jnp.float32),
            pltpu.SemaphoreType.DMA,
            pltpu.SemaphoreType.DMA,
            pltpu.SemaphoreType.DMA,
            pltpu.SemaphoreType.DMA,
        ],
        compiler_params=pltpu.CompilerParams(
            use_tc_tiling_on_sc=False, needs_layout_passes=False
        ),
    )
    return f(t, bb_flat)


_PERM = []
for _h in range(DIM // 32):
    for _j in range(16):
        _PERM.extend((_h * 32 + _j, _h * 32 + 16 + _j))


def kernel(gt_bboxes, x_table, y_table, W, b):
    B, N, _ = gt_bboxes.shape
    perm = jnp.array(_PERM, jnp.int32)
    t = _build_table(x_table, y_table, W[perm, :], b[perm])
    bb_full = gt_bboxes.reshape(B * N * 4)
    outs = []
    for lo, hi, chunk in SPLITS:
        rows_h = (hi - lo) * N
        bbh = lax.slice(bb_full, (lo * N * 4,), (hi * N * 4,))
        outs.append(_lookup(t, bbh, rows_h, chunk).reshape(hi - lo, N, DIM))
    return jnp.concatenate(outs, axis=0)


# speedup vs baseline: 1.1093x; 1.1093x over previous
import functools

import jax
import jax.numpy as jnp
from jax import lax
from jax.experimental import pallas as pl
from jax.experimental.pallas import tpu as pltpu
from jax.experimental.pallas import tpu_sc as plsc

MAX_POS = 1024
DIM = 64

try:
    _INFO = plsc.get_sparse_core_info()
    NC, NS, L = _INFO.num_cores, _INFO.num_subcores, _INFO.num_lanes
except Exception:
    NC, NS, L = 2, 16, 16
NW = NC * NS

GBLK = 128
SPLITS = ((0, 2048, 64), (2048, 4096, 64))


def _table_body(x_ref, y_ref, w_ref, b_ref, t_ref):
    wt = w_ref[...].T
    bias = b_ref[...] * 0.25
    t_ref[0:MAX_POS, :] = (
        jnp.dot(x_ref[...], wt, preferred_element_type=jnp.float32) + bias
    ).astype(jnp.bfloat16)
    t_ref[MAX_POS : 2 * MAX_POS, :] = (
        jnp.dot(y_ref[...], wt, preferred_element_type=jnp.float32) + bias
    ).astype(jnp.bfloat16)


def _build_table(x_table, y_table, W, b):
    return pl.pallas_call(
        _table_body,
        out_shape=jax.ShapeDtypeStruct((2 * MAX_POS, DIM), jnp.bfloat16),
    )(x_table, y_table, W, b.reshape(1, DIM))


def _sc_body(
    rows_total,
    CHUNK,
    t_hbm,
    bb_hbm,
    out_hbm,
    bb_v,
    idx_v,
    rows_v,
    out_v,
    sem0,
    sem1,
    semw0,
    semw1,
):
    rw = rows_total // NW
    n_chunks = rw // CHUNK
    wid = lax.axis_index("s") * NC + lax.axis_index("c")
    base_row = wid * rw
    sems = (sem0, sem1)
    semws = (semw0, semw1)
    nj = 4 * CHUNK // GBLK

    offs = (lax.iota(jnp.int32, L) % 2) * MAX_POS

    def stage_and_fire(c, p):
        row0 = base_row + c * CHUNK
        pltpu.sync_copy(bb_hbm.at[pl.ds(row0 * 4, 4 * CHUNK)], bb_v)
        for j in range(nj):
            for i in range(GBLK // L):
                v = bb_v[pl.ds(j * GBLK + i * L, L)]
                f = jnp.clip(v * float(MAX_POS), 0.0, float(MAX_POS - 1))
                idx_v[p, j, pl.ds(i * L, L)] = f.astype(jnp.int32) + offs
        for j in range(nj):
            pltpu.async_copy(
                t_hbm.at[idx_v.at[p, j]],
                rows_v.at[p, pl.ds(j * GBLK, GBLK)],
                sems[p],
            )

    def consume(c, p, k):
        row0 = base_row + c * CHUNK
        for j in range(nj):
            pltpu.make_async_copy(
                t_hbm.at[idx_v.at[p, j]],
                rows_v.at[p, pl.ds(j * GBLK, GBLK)],
                sems[p],
            ).wait()

        @pl.when(k > 0)
        def _():
            prev0 = base_row + (c - 2) * CHUNK
            pltpu.make_async_copy(
                out_v.at[p], out_hbm.at[pl.ds(prev0, CHUNK)], semws[p]
            ).wait()

        @plsc.parallel_loop(0, CHUNK, 1, unroll=8)
        def sum_body(r):
            for h in range(DIM // (2 * L)):
                ds = pl.ds(h * 2 * L, 2 * L)
                a = [None] * 4
                bvs = [None] * 4
                for i in range(4):
                    a[i], bvs[i] = plsc.unpack(
                        rows_v[p, 4 * r + i, ds], format=plsc.PackFormat.INTERLEAVED
                    )
                sa = (a[0] + a[1]) + (a[2] + a[3])
                sb = (bvs[0] + bvs[1]) + (bvs[2] + bvs[3])
                out_v[p, r, pl.ds(h * 2 * L, L)] = jnp.maximum(sa, 0.0)
                out_v[p, r, pl.ds(h * 2 * L + L, L)] = jnp.maximum(sb, 0.0)

        pltpu.async_copy(out_v.at[p], out_hbm.at[pl.ds(row0, CHUNK)], semws[p])

    stage_and_fire(0, 0)

    def pair_body(k, carry):
        for p in range(2):
            c = 2 * k + p

            @pl.when(c + 1 < n_chunks)
            def _():
                stage_and_fire(c + 1, 1 - p)

            consume(c, p, k)
        return carry

    lax.fori_loop(0, n_chunks // 2, pair_body, 0)

    for p in range(2):
        last = base_row + (n_chunks - 2 + p) * CHUNK
        pltpu.make_async_copy(
            out_v.at[p], out_hbm.at[pl.ds(last, CHUNK)], semws[p]
        ).wait()


def _lookup(t, bb_flat, rows_total, CHUNK):
    mesh = plsc.VectorSubcoreMesh(
        core_axis_name="c", subcore_axis_name="s", num_cores=NC, num_subcores=NS
    )
    f = pl.kernel(
        functools.partial(_sc_body, rows_total, CHUNK),
        out_type=jax.ShapeDtypeStruct((rows_total, DIM), jnp.float32),
        mesh=mesh,
        scratch_types=[
            pltpu.VMEM((4 * CHUNK,), jnp.float32),
            pltpu.VMEM((2, 4 * CHUNK // GBLK, GBLK), jnp.int32),
            pltpu.VMEM((2, 4 * CHUNK, DIM), jnp.bfloat16),
            pltpu.VMEM((2, CHUNK, DIM), jnp.float32),
            pltpu.SemaphoreType.DMA,
            pltpu.SemaphoreType.DMA,
            pltpu.SemaphoreType.DMA,
            pltpu.SemaphoreType.DMA,
        ],
        compiler_params=pltpu.CompilerParams(
            use_tc_tiling_on_sc=False, needs_layout_passes=False
        ),
    )
    return f(t, bb_flat)


_PERM = []
for _h in range(DIM // 32):
    for _j in range(16):
        _PERM.extend((_h * 32 + _j, _h * 32 + 16 + _j))


def kernel(gt_bboxes, x_table, y_table, W, b):
    B, N, _ = gt_bboxes.shape
    perm = jnp.array(_PERM, jnp.int32)
    t = _build_table(x_table, y_table, W[perm, :], b[perm])
    outs = []
    for lo, hi, chunk in SPLITS:
        rows_h = (hi - lo) * N
        bbh = gt_bboxes[lo:hi].reshape(rows_h * 4)
        outs.append(_lookup(t, bbh, rows_h, chunk).reshape(hi - lo, N, DIM))
    return jnp.concatenate(outs, axis=0)


# speedup vs baseline: 1.1228x; 1.0122x over previous
import functools

import jax
import jax.numpy as jnp
from jax import lax
from jax.experimental import pallas as pl
from jax.experimental.pallas import tpu as pltpu
from jax.experimental.pallas import tpu_sc as plsc

MAX_POS = 1024
DIM = 64

try:
    _INFO = plsc.get_sparse_core_info()
    NC, NS, L = _INFO.num_cores, _INFO.num_subcores, _INFO.num_lanes
except Exception:
    NC, NS, L = 2, 16, 16
NW = NC * NS

GBLK = 80
SPLITS = ((0, 2048, 80), (2048, 4096, 80))


def _table_body(x_ref, y_ref, w_ref, b_ref, t_ref):
    wt = w_ref[...].T
    bias = b_ref[...] * 0.25
    t_ref[0:MAX_POS, :] = (
        jnp.dot(x_ref[...], wt, preferred_element_type=jnp.float32) + bias
    ).astype(jnp.bfloat16)
    t_ref[MAX_POS : 2 * MAX_POS, :] = (
        jnp.dot(y_ref[...], wt, preferred_element_type=jnp.float32) + bias
    ).astype(jnp.bfloat16)


def _build_table(x_table, y_table, W, b):
    return pl.pallas_call(
        _table_body,
        out_shape=jax.ShapeDtypeStruct((2 * MAX_POS, DIM), jnp.bfloat16),
    )(x_table, y_table, W, b.reshape(1, DIM))


def _sc_body(
    rows_total,
    CHUNK,
    t_hbm,
    bb_hbm,
    out_hbm,
    bb_v,
    idx_v,
    rows_v,
    out_v,
    sem0,
    sem1,
    semw0,
    semw1,
):
    rw = rows_total // NW
    n_chunks = rw // CHUNK
    wid = lax.axis_index("s") * NC + lax.axis_index("c")
    base_row = wid * rw
    sems = (sem0, sem1)
    semws = (semw0, semw1)
    nj = 4 * CHUNK // GBLK

    offs = (lax.iota(jnp.int32, L) % 2) * MAX_POS

    def stage_and_fire(c, p):
        row0 = base_row + c * CHUNK
        pltpu.sync_copy(bb_hbm.at[pl.ds(row0 * 4, 4 * CHUNK)], bb_v)
        for j in range(nj):
            for i in range(GBLK // L):
                v = bb_v[pl.ds(j * GBLK + i * L, L)]
                f = jnp.clip(v * float(MAX_POS), 0.0, float(MAX_POS - 1))
                idx_v[p, j, pl.ds(i * L, L)] = f.astype(jnp.int32) + offs
        for j in range(nj):
            pltpu.async_copy(
                t_hbm.at[idx_v.at[p, j]],
                rows_v.at[p, pl.ds(j * GBLK, GBLK)],
                sems[p],
            )

    def consume(c, p, k):
        row0 = base_row + c * CHUNK
        for j in range(nj):
            pltpu.make_async_copy(
                t_hbm.at[idx_v.at[p, j]],
                rows_v.at[p, pl.ds(j * GBLK, GBLK)],
                sems[p],
            ).wait()

        @pl.when(k > 0)
        def _():
            prev0 = base_row + (c - 2) * CHUNK
            pltpu.make_async_copy(
                out_v.at[p], out_hbm.at[pl.ds(prev0, CHUNK)], semws[p]
            ).wait()

        @plsc.parallel_loop(0, CHUNK, 1, unroll=8)
        def sum_body(r):
            for h in range(DIM // (2 * L)):
                ds = pl.ds(h * 2 * L, 2 * L)
                a = [None] * 4
                bvs = [None] * 4
                for i in range(4):
                    a[i], bvs[i] = plsc.unpack(
                        rows_v[p, 4 * r + i, ds], format=plsc.PackFormat.INTERLEAVED
                    )
                sa = (a[0] + a[1]) + (a[2] + a[3])
                sb = (bvs[0] + bvs[1]) + (bvs[2] + bvs[3])
                out_v[p, r, pl.ds(h * 2 * L, L)] = jnp.maximum(sa, 0.0)
                out_v[p, r, pl.ds(h * 2 * L + L, L)] = jnp.maximum(sb, 0.0)

        pltpu.async_copy(out_v.at[p], out_hbm.at[pl.ds(row0, CHUNK)], semws[p])

    stage_and_fire(0, 0)

    def pair_body(k, carry):
        for p in range(2):
            c = 2 * k + p

            @pl.when(c + 1 < n_chunks)
            def _():
                stage_and_fire(c + 1, 1 - p)

            consume(c, p, k)
        return carry

    lax.fori_loop(0, n_chunks // 2, pair_body, 0)

    for p in range(2):
        last = base_row + (n_chunks - 2 + p) * CHUNK
        pltpu.make_async_copy(
            out_v.at[p], out_hbm.at[pl.ds(last, CHUNK)], semws[p]
        ).wait()


def _lookup(t, bb_flat, rows_total, CHUNK):
    mesh = plsc.VectorSubcoreMesh(
        core_axis_name="c", subcore_axis_name="s", num_cores=NC, num_subcores=NS
    )
    f = pl.kernel(
        functools.partial(_sc_body, rows_total, CHUNK),
        out_type=jax.ShapeDtypeStruct((rows_total, DIM), jnp.float32),
        mesh=mesh,
        scratch_types=[
            pltpu.VMEM((4 * CHUNK,), jnp.float32),
            pltpu.VMEM((2, 4 * CHUNK // GBLK, GBLK), jnp.int32),
            pltpu.VMEM((2, 4 * CHUNK, DIM), jnp.bfloat16),
            pltpu.VMEM((2, CHUNK, DIM), jnp.float32),
            pltpu.SemaphoreType.DMA,
            pltpu.SemaphoreType.DMA,
            pltpu.SemaphoreType.DMA,
            pltpu.SemaphoreType.DMA,
        ],
        compiler_params=pltpu.CompilerParams(
            use_tc_tiling_on_sc=False, needs_layout_passes=False
        ),
    )
    return f(t, bb_flat)


_PERM = []
for _h in range(DIM // 32):
    for _j in range(16):
        _PERM.extend((_h * 32 + _j, _h * 32 + 16 + _j))


def kernel(gt_bboxes, x_table, y_table, W, b):
    B, N, _ = gt_bboxes.shape
    perm = jnp.array(_PERM, jnp.int32)
    t = _build_table(x_table, y_table, W[perm, :], b[perm])
    outs = []
    for lo, hi, chunk in SPLITS:
        rows_h = (hi - lo) * N
        bbh = gt_bboxes[lo:hi].reshape(rows_h * 4)
        outs.append(_lookup(t, bbh, rows_h, chunk).reshape(hi - lo, N, DIM))
    return jnp.concatenate(outs, axis=0)


# speedup vs baseline: 1.1317x; 1.0079x over previous
import functools

import jax
import jax.numpy as jnp
from jax import lax
from jax.experimental import pallas as pl
from jax.experimental.pallas import tpu as pltpu
from jax.experimental.pallas import tpu_sc as plsc

MAX_POS = 1024
DIM = 64

try:
    _INFO = plsc.get_sparse_core_info()
    NC, NS, L = _INFO.num_cores, _INFO.num_subcores, _INFO.num_lanes
except Exception:
    NC, NS, L = 2, 16, 16
NW = NC * NS

GBLK = 80
SPLITS = ((0, 2048, 160), (2048, 4096, 160))


def _table_body(x_ref, y_ref, w_ref, b_ref, t_ref):
    wt = w_ref[...].T
    bias = b_ref[...] * 0.25
    t_ref[0:MAX_POS, :] = (
        jnp.dot(x_ref[...], wt, preferred_element_type=jnp.float32) + bias
    ).astype(jnp.bfloat16)
    t_ref[MAX_POS : 2 * MAX_POS, :] = (
        jnp.dot(y_ref[...], wt, preferred_element_type=jnp.float32) + bias
    ).astype(jnp.bfloat16)


def _build_table(x_table, y_table, W, b):
    return pl.pallas_call(
        _table_body,
        out_shape=jax.ShapeDtypeStruct((2 * MAX_POS, DIM), jnp.bfloat16),
    )(x_table, y_table, W, b.reshape(1, DIM))


def _sc_body(
    rows_total,
    CHUNK,
    t_hbm,
    bb_hbm,
    out_hbm,
    bb_v,
    idx_v,
    rows_v,
    out_v,
    sem0,
    sem1,
    semw0,
    semw1,
):
    rw = rows_total // NW
    n_chunks = rw // CHUNK
    wid = lax.axis_index("s") * NC + lax.axis_index("c")
    base_row = wid * rw
    sems = (sem0, sem1)
    semws = (semw0, semw1)
    nj = 4 * CHUNK // GBLK

    offs = (lax.iota(jnp.int32, L) % 2) * MAX_POS

    def stage_and_fire(c, p):
        row0 = base_row + c * CHUNK
        pltpu.sync_copy(bb_hbm.at[pl.ds(row0 * 4, 4 * CHUNK)], bb_v)
        for j in range(nj):
            for i in range(GBLK // L):
                v = bb_v[pl.ds(j * GBLK + i * L, L)]
                f = jnp.clip(v * float(MAX_POS), 0.0, float(MAX_POS - 1))
                idx_v[p, j, pl.ds(i * L, L)] = f.astype(jnp.int32) + offs
        for j in range(nj):
            pltpu.async_copy(
                t_hbm.at[idx_v.at[p, j]],
                rows_v.at[p, pl.ds(j * GBLK, GBLK)],
                sems[p],
            )

    def consume(c, p, k):
        row0 = base_row + c * CHUNK
        for j in range(nj):
            pltpu.make_async_copy(
                t_hbm.at[idx_v.at[p, j]],
                rows_v.at[p, pl.ds(j * GBLK, GBLK)],
                sems[p],
            ).wait()

        @pl.when(k > 0)
        def _():
            prev0 = base_row + (c - 2) * CHUNK
            pltpu.make_async_copy(
                out_v.at[p], out_hbm.at[pl.ds(prev0, CHUNK)], semws[p]
            ).wait()

        @plsc.parallel_loop(0, CHUNK, 1, unroll=8)
        def sum_body(r):
            for h in range(DIM // (2 * L)):
                ds = pl.ds(h * 2 * L, 2 * L)
                a = [None] * 4
                bvs = [None] * 4
                for i in range(4):
                    a[i], bvs[i] = plsc.unpack(
                        rows_v[p, 4 * r + i, ds], format=plsc.PackFormat.INTERLEAVED
                    )
                sa = (a[0] + a[1]) + (a[2] + a[3])
                sb = (bvs[0] + bvs[1]) + (bvs[2] + bvs[3])
                out_v[p, r, pl.ds(h * 2 * L, L)] = jnp.maximum(sa, 0.0)
                out_v[p, r, pl.ds(h * 2 * L + L, L)] = jnp.maximum(sb, 0.0)

        pltpu.async_copy(out_v.at[p], out_hbm.at[pl.ds(row0, CHUNK)], semws[p])

    stage_and_fire(0, 0)

    def pair_body(k, carry):
        for p in range(2):
            c = 2 * k + p

            @pl.when(c + 1 < n_chunks)
            def _():
                stage_and_fire(c + 1, 1 - p)

            consume(c, p, k)
        return carry

    lax.fori_loop(0, n_chunks // 2, pair_body, 0)

    for p in range(2):
        last = base_row + (n_chunks - 2 + p) * CHUNK
        pltpu.make_async_copy(
            out_v.at[p], out_hbm.at[pl.ds(last, CHUNK)], semws[p]
        ).wait()


def _lookup(t, bb_flat, rows_total, CHUNK):
    mesh = plsc.VectorSubcoreMesh(
        core_axis_name="c", subcore_axis_name="s", num_cores=NC, num_subcores=NS
    )
    f = pl.kernel(
        functools.partial(_sc_body, rows_total, CHUNK),
        out_type=jax.ShapeDtypeStruct((rows_total, DIM), jnp.float32),
        mesh=mesh,
        scratch_types=[
            pltpu.VMEM((4 * CHUNK,), jnp.float32),
            pltpu.VMEM((2, 4 * CHUNK // GBLK, GBLK), jnp.int32),
            pltpu.VMEM((2, 4 * CHUNK, DIM), jnp.bfloat16),
            pltpu.VMEM((2, CHUNK, DIM), jnp.float32),
            pltpu.SemaphoreType.DMA,
            pltpu.SemaphoreType.DMA,
            pltpu.SemaphoreType.DMA,
            pltpu.SemaphoreType.DMA,
        ],
        compiler_params=pltpu.CompilerParams(
            use_tc_tiling_on_sc=False, needs_layout_passes=False
        ),
    )
    return f(t, bb_flat)


_PERM = []
for _h in range(DIM // 32):
    for _j in range(16):
        _PERM.extend((_h * 32 + _j, _h * 32 + 16 + _j))


def kernel(gt_bboxes, x_table, y_table, W, b):
    B, N, _ = gt_bboxes.shape
    perm = jnp.array(_PERM, jnp.int32)
    t = _build_table(x_table, y_table, W[perm, :], b[perm])
    outs = []
    for lo, hi, chunk in SPLITS:
        rows_h = (hi - lo) * N
        bbh = gt_bboxes[lo:hi].reshape(rows_h * 4)
        outs.append(_lookup(t, bbh, rows_h, chunk).reshape(hi - lo, N, DIM))
    return jnp.concatenate(outs, axis=0)
